# baseline (speedup 1.0000x reference)
import jax
import jax.numpy as jnp
from jax.experimental import pallas as pl
from jax.experimental.pallas import tpu as pltpu

_S = 4
_K = 1024
_D = 256
_T = 512


def _mlp_body(cb_ref, w1_ref, b1_ref, w2_ref, b2_ref,
              cbeff_ref, bytes_ref):
    for i in range(_S):
        cb = cb_ref[i]
        h = jnp.maximum(jnp.dot(cb, w1_ref[i]) + b1_ref[i], 0.0)
        cb_eff = jnp.dot(h, w2_ref[i]) + b2_ref[i]
        cbeff_ref[i] = cb_eff
        u = jax.lax.bitcast_convert_type(cb_eff, jnp.int32)
        for j in range(4):
            bytes_ref[i, j] = ((u >> (8 * j)) & 0xFF).astype(jnp.bfloat16)


_P = 2
_TP = _T // _P


def _fused_body(z_ref, r20_ref, cb_ref, c2_ref, bytes_ref, out_ref):
    lanes = jax.lax.broadcasted_iota(jnp.int32, (_TP, _K), 1)
    zs = [z_ref[p * _TP:(p + 1) * _TP] for p in range(_P)]
    residual = list(zs)
    total = [jnp.zeros_like(zs[p]) for p in range(_P)]
    for i in range(_S):
        for p in range(_P):
            if i == 0:
                r2 = r20_ref[p * _TP:(p + 1) * _TP]
            else:
                r2 = jnp.sum(residual[p] * residual[p],
                             axis=-1, keepdims=True)
            e = jax.lax.dot_general(residual[p], cb_ref[i],
                                    (((1,), (1,)), ((), ())))
            d = (r2 - 2.0 * e) + c2_ref[i]
            mn = jnp.min(d, axis=-1, keepdims=True)
            idx = jnp.min(jnp.where(d == mn, lanes, _K), axis=-1)
            onehot = (lanes == idx[:, None]).astype(jnp.bfloat16)
            qb = [jax.lax.dot_general(onehot, bytes_ref[i, j],
                                      (((1,), (0,)), ((), ())),
                                      preferred_element_type=jnp.float32)
                  for j in range(4)]
            qi = (qb[0].astype(jnp.int32)
                  | (qb[1].astype(jnp.int32) << 8)
                  | (qb[2].astype(jnp.int32) << 16)
                  | (qb[3].astype(jnp.int32) << 24))
            q = jax.lax.bitcast_convert_type(qi, jnp.float32)
            residual[p] = residual[p] - q
            total[p] = total[p] + q
    for p in range(_P):
        out_ref[p * _TP:(p + 1) * _TP] = zs[p] + (total[p] - zs[p])


def kernel(z, codebooks, W1, b1, W2, b2):
    B, N, D = z.shape
    nt = B * N

    cb_eff, planes = pl.pallas_call(
        _mlp_body,
        out_shape=[
            jax.ShapeDtypeStruct((_S, _K, _D), jnp.float32),
            jax.ShapeDtypeStruct((_S, 4, _K, _D), jnp.bfloat16),
        ],
    )(codebooks, W1, b1.reshape(_S, 1, _D), W2, b2.reshape(_S, 1, _D))
    c2 = jnp.sum(cb_eff ** 2, axis=-1).reshape(_S, 1, _K)
    r20 = jnp.sum(z ** 2, axis=-1).reshape(nt, 1)

    out = pl.pallas_call(
        _fused_body,
        grid=(nt // _T,),
        in_specs=[
            pl.BlockSpec((_T, _D), lambda t: (t, 0)),
            pl.BlockSpec((_T, 1), lambda t: (t, 0)),
            pl.BlockSpec((_S, _K, _D), lambda t: (0, 0, 0)),
            pl.BlockSpec((_S, 1, _K), lambda t: (0, 0, 0)),
            pl.BlockSpec((_S, 4, _K, _D), lambda t: (0, 0, 0, 0)),
        ],
        out_specs=pl.BlockSpec((_T, _D), lambda t: (t, 0)),
        out_shape=jax.ShapeDtypeStruct((nt, D), jnp.float32),
    )(z.reshape(nt, D), r20, cb_eff, c2, planes)
    return out.reshape(B, N, D)


# speedup vs baseline: 1.2010x; 1.2010x over previous
import jax
import jax.numpy as jnp
from jax.experimental import pallas as pl
from jax.experimental.pallas import tpu as pltpu

_S = 4
_K = 1024
_D = 256
_T = 1024


def _mlp_body(cb_ref, w1_ref, b1_ref, w2_ref, b2_ref,
              cbeff_ref, bytes_ref):
    for i in range(_S):
        cb = cb_ref[i]
        h = jnp.maximum(jnp.dot(cb, w1_ref[i]) + b1_ref[i], 0.0)
        cb_eff = jnp.dot(h, w2_ref[i]) + b2_ref[i]
        cbeff_ref[i] = cb_eff
        u = jax.lax.bitcast_convert_type(cb_eff, jnp.int32)
        for j in range(4):
            bytes_ref[i, j] = ((u >> (8 * j)) & 0xFF).astype(jnp.bfloat16)


_P = 2
_TP = _T // _P


def _fused_body(z_ref, r20_ref, cb_ref, c2_ref, bytes_ref, out_ref):
    lanes = jax.lax.broadcasted_iota(jnp.int32, (_TP, _K), 1)
    zs = [z_ref[p * _TP:(p + 1) * _TP] for p in range(_P)]
    residual = list(zs)
    total = [jnp.zeros_like(zs[p]) for p in range(_P)]
    for i in range(_S):
        for p in range(_P):
            if i == 0:
                r2 = r20_ref[p * _TP:(p + 1) * _TP]
            else:
                r2 = jnp.sum(residual[p] * residual[p],
                             axis=-1, keepdims=True)
            e = jax.lax.dot_general(residual[p], cb_ref[i],
                                    (((1,), (1,)), ((), ())))
            d = (r2 - 2.0 * e) + c2_ref[i]
            mn = jnp.min(d, axis=-1, keepdims=True)
            idx = jnp.min(jnp.where(d == mn, lanes, _K), axis=-1)
            onehot = (lanes == idx[:, None]).astype(jnp.bfloat16)
            qb = [jax.lax.dot_general(onehot, bytes_ref[i, j],
                                      (((1,), (0,)), ((), ())),
                                      preferred_element_type=jnp.float32)
                  for j in range(4)]
            qi = (qb[0].astype(jnp.int32)
                  | (qb[1].astype(jnp.int32) << 8)
                  | (qb[2].astype(jnp.int32) << 16)
                  | (qb[3].astype(jnp.int32) << 24))
            q = jax.lax.bitcast_convert_type(qi, jnp.float32)
            residual[p] = residual[p] - q
            total[p] = total[p] + q
    for p in range(_P):
        out_ref[p * _TP:(p + 1) * _TP] = zs[p] + (total[p] - zs[p])


def kernel(z, codebooks, W1, b1, W2, b2):
    B, N, D = z.shape
    nt = B * N

    cb_eff, planes = pl.pallas_call(
        _mlp_body,
        out_shape=[
            jax.ShapeDtypeStruct((_S, _K, _D), jnp.float32),
            jax.ShapeDtypeStruct((_S, 4, _K, _D), jnp.bfloat16),
        ],
    )(codebooks, W1, b1.reshape(_S, 1, _D), W2, b2.reshape(_S, 1, _D))
    c2 = jnp.sum(cb_eff ** 2, axis=-1).reshape(_S, 1, _K)
    r20 = jnp.sum(z ** 2, axis=-1).reshape(nt, 1)

    out = pl.pallas_call(
        _fused_body,
        grid=(nt // _T,),
        in_specs=[
            pl.BlockSpec((_T, _D), lambda t: (t, 0)),
            pl.BlockSpec((_T, 1), lambda t: (t, 0)),
            pl.BlockSpec((_S, _K, _D), lambda t: (0, 0, 0)),
            pl.BlockSpec((_S, 1, _K), lambda t: (0, 0, 0)),
            pl.BlockSpec((_S, 4, _K, _D), lambda t: (0, 0, 0, 0)),
        ],
        out_specs=pl.BlockSpec((_T, _D), lambda t: (t, 0)),
        out_shape=jax.ShapeDtypeStruct((nt, D), jnp.float32),
    )(z.reshape(nt, D), r20, cb_eff, c2, planes)
    return out.reshape(B, N, D)


# speedup vs baseline: 1.2050x; 1.0033x over previous
import jax
import jax.numpy as jnp
from jax.experimental import pallas as pl
from jax.experimental.pallas import tpu as pltpu

_S = 4
_K = 1024
_D = 256
_T = 2048


def _mlp_body(cb_ref, w1_ref, b1_ref, w2_ref, b2_ref,
              cbeff_ref, bytes_ref):
    for i in range(_S):
        cb = cb_ref[i]
        h = jnp.maximum(jnp.dot(cb, w1_ref[i]) + b1_ref[i], 0.0)
        cb_eff = jnp.dot(h, w2_ref[i]) + b2_ref[i]
        cbeff_ref[i] = cb_eff
        u = jax.lax.bitcast_convert_type(cb_eff, jnp.int32)
        for j in range(4):
            bytes_ref[i, j] = ((u >> (8 * j)) & 0xFF).astype(jnp.bfloat16)


_P = 4
_TP = _T // _P


def _fused_body(z_ref, r20_ref, cb_ref, c2_ref, bytes_ref, out_ref):
    lanes = jax.lax.broadcasted_iota(jnp.int32, (_TP, _K), 1)
    zs = [z_ref[p * _TP:(p + 1) * _TP] for p in range(_P)]
    residual = list(zs)
    total = [jnp.zeros_like(zs[p]) for p in range(_P)]
    for i in range(_S):
        for p in range(_P):
            if i == 0:
                r2 = r20_ref[p * _TP:(p + 1) * _TP]
            else:
                r2 = jnp.sum(residual[p] * residual[p],
                             axis=-1, keepdims=True)
            e = jax.lax.dot_general(residual[p], cb_ref[i],
                                    (((1,), (1,)), ((), ())))
            d = (r2 - 2.0 * e) + c2_ref[i]
            mn = jnp.min(d, axis=-1, keepdims=True)
            idx = jnp.min(jnp.where(d == mn, lanes, _K), axis=-1)
            onehot = (lanes == idx[:, None]).astype(jnp.bfloat16)
            qb = [jax.lax.dot_general(onehot, bytes_ref[i, j],
                                      (((1,), (0,)), ((), ())),
                                      preferred_element_type=jnp.float32)
                  for j in range(4)]
            qi = (qb[0].astype(jnp.int32)
                  | (qb[1].astype(jnp.int32) << 8)
                  | (qb[2].astype(jnp.int32) << 16)
                  | (qb[3].astype(jnp.int32) << 24))
            q = jax.lax.bitcast_convert_type(qi, jnp.float32)
            residual[p] = residual[p] - q
            total[p] = total[p] + q
    for p in range(_P):
        out_ref[p * _TP:(p + 1) * _TP] = zs[p] + (total[p] - zs[p])


def kernel(z, codebooks, W1, b1, W2, b2):
    B, N, D = z.shape
    nt = B * N

    cb_eff, planes = pl.pallas_call(
        _mlp_body,
        out_shape=[
            jax.ShapeDtypeStruct((_S, _K, _D), jnp.float32),
            jax.ShapeDtypeStruct((_S, 4, _K, _D), jnp.bfloat16),
        ],
    )(codebooks, W1, b1.reshape(_S, 1, _D), W2, b2.reshape(_S, 1, _D))
    c2 = jnp.sum(cb_eff ** 2, axis=-1).reshape(_S, 1, _K)
    r20 = jnp.sum(z ** 2, axis=-1).reshape(nt, 1)

    out = pl.pallas_call(
        _fused_body,
        grid=(nt // _T,),
        in_specs=[
            pl.BlockSpec((_T, _D), lambda t: (t, 0)),
            pl.BlockSpec((_T, 1), lambda t: (t, 0)),
            pl.BlockSpec((_S, _K, _D), lambda t: (0, 0, 0)),
            pl.BlockSpec((_S, 1, _K), lambda t: (0, 0, 0)),
            pl.BlockSpec((_S, 4, _K, _D), lambda t: (0, 0, 0, 0)),
        ],
        out_specs=pl.BlockSpec((_T, _D), lambda t: (t, 0)),
        out_shape=jax.ShapeDtypeStruct((nt, D), jnp.float32),
    )(z.reshape(nt, D), r20, cb_eff, c2, planes)
    return out.reshape(B, N, D)


# speedup vs baseline: 1.2532x; 1.0399x over previous
import jax
import jax.numpy as jnp
from jax.experimental import pallas as pl
from jax.experimental.pallas import tpu as pltpu

_S = 4
_K = 1024
_D = 256
_T = 2048


def _mlp_body(cb_ref, w1_ref, b1_ref, w2_ref, b2_ref,
              cbeff_ref, bytes_ref, c2_ref):
    for i in range(_S):
        cb = cb_ref[i]
        h = jnp.maximum(jnp.dot(cb, w1_ref[i]) + b1_ref[i], 0.0)
        cb_eff = jnp.dot(h, w2_ref[i]) + b2_ref[i]
        cbeff_ref[i] = cb_eff
        c2_ref[i] = jnp.sum(cb_eff * cb_eff, axis=-1)[None, :]
        u = jax.lax.bitcast_convert_type(cb_eff, jnp.int32)
        for j in range(4):
            bytes_ref[i, j] = ((u >> (8 * j)) & 0xFF).astype(jnp.bfloat16)


_P = 4
_TP = _T // _P


def _fused_body(z_ref, cb_ref, c2_ref, bytes_ref, out_ref):
    lanes = jax.lax.broadcasted_iota(jnp.int32, (_TP, _K), 1)
    zs = [z_ref[p * _TP:(p + 1) * _TP] for p in range(_P)]
    residual = list(zs)
    total = [jnp.zeros_like(zs[p]) for p in range(_P)]
    for i in range(_S):
        for p in range(_P):
            r2 = jnp.sum(residual[p] * residual[p],
                         axis=-1, keepdims=True)
            e = jax.lax.dot_general(residual[p], cb_ref[i],
                                    (((1,), (1,)), ((), ())))
            d = (r2 - 2.0 * e) + c2_ref[i]
            mn = jnp.min(d, axis=-1, keepdims=True)
            idx = jnp.min(jnp.where(d == mn, lanes, _K), axis=-1)
            onehot = (lanes == idx[:, None]).astype(jnp.bfloat16)
            qb = [jax.lax.dot_general(onehot, bytes_ref[i, j],
                                      (((1,), (0,)), ((), ())),
                                      preferred_element_type=jnp.float32)
                  for j in range(4)]
            qi = (qb[0].astype(jnp.int32)
                  | (qb[1].astype(jnp.int32) << 8)
                  | (qb[2].astype(jnp.int32) << 16)
                  | (qb[3].astype(jnp.int32) << 24))
            q = jax.lax.bitcast_convert_type(qi, jnp.float32)
            residual[p] = residual[p] - q
            total[p] = total[p] + q
    for p in range(_P):
        out_ref[p * _TP:(p + 1) * _TP] = zs[p] + (total[p] - zs[p])


def kernel(z, codebooks, W1, b1, W2, b2):
    B, N, D = z.shape
    nt = B * N

    cb_eff, planes, c2 = pl.pallas_call(
        _mlp_body,
        out_shape=[
            jax.ShapeDtypeStruct((_S, _K, _D), jnp.float32),
            jax.ShapeDtypeStruct((_S, 4, _K, _D), jnp.bfloat16),
            jax.ShapeDtypeStruct((_S, 1, _K), jnp.float32),
        ],
    )(codebooks, W1, b1.reshape(_S, 1, _D), W2, b2.reshape(_S, 1, _D))

    out = pl.pallas_call(
        _fused_body,
        grid=(nt // _T,),
        in_specs=[
            pl.BlockSpec((_T, _D), lambda t: (t, 0)),
            pl.BlockSpec((_S, _K, _D), lambda t: (0, 0, 0)),
            pl.BlockSpec((_S, 1, _K), lambda t: (0, 0, 0)),
            pl.BlockSpec((_S, 4, _K, _D), lambda t: (0, 0, 0, 0)),
        ],
        out_specs=pl.BlockSpec((_T, _D), lambda t: (t, 0)),
        out_shape=jax.ShapeDtypeStruct((nt, D), jnp.float32),
    )(z.reshape(nt, D), cb_eff, c2, planes)
    return out.reshape(B, N, D)


# speedup vs baseline: 1.2721x; 1.0151x over previous
import jax
import jax.numpy as jnp
from jax.experimental import pallas as pl
from jax.experimental.pallas import tpu as pltpu

_S = 4
_K = 1024
_D = 256
_T = 2048


def _mlp_body(cb_ref, w1_ref, b1_ref, w2_ref, b2_ref,
              cb2_ref, bytes_ref, c2_ref):
    for i in range(_S):
        cb = cb_ref[i]
        h = jnp.maximum(jnp.dot(cb, w1_ref[i]) + b1_ref[i], 0.0)
        cb_eff = jnp.dot(h, w2_ref[i]) + b2_ref[i]
        cb2_ref[i] = cb_eff + cb_eff
        c2_ref[i] = jnp.sum(cb_eff * cb_eff, axis=-1)[None, :]
        u = jax.lax.bitcast_convert_type(cb_eff, jnp.int32)
        for j in range(4):
            bytes_ref[i, j] = ((u >> (8 * j)) & 0xFF).astype(jnp.bfloat16)


_P = 4
_TP = _T // _P


def _fused_body(z_ref, cb_ref, c2_ref, bytes_ref, out_ref):
    lanes = jax.lax.broadcasted_iota(jnp.int32, (_TP, _K), 1)
    zs = [z_ref[p * _TP:(p + 1) * _TP] for p in range(_P)]
    residual = list(zs)
    total = [jnp.zeros_like(zs[p]) for p in range(_P)]
    for i in range(_S):
        for p in range(_P):
            r2 = jnp.sum(residual[p] * residual[p],
                         axis=-1, keepdims=True)
            e2 = jax.lax.dot_general(residual[p], cb_ref[i],
                                     (((1,), (1,)), ((), ())))
            d = (r2 - e2) + c2_ref[i]
            mn = jnp.min(d, axis=-1, keepdims=True)
            idx = jnp.min(jnp.where(d == mn, lanes, _K), axis=-1)
            onehot = (lanes == idx[:, None]).astype(jnp.bfloat16)
            qb = [jax.lax.dot_general(onehot, bytes_ref[i, j],
                                      (((1,), (0,)), ((), ())),
                                      preferred_element_type=jnp.float32)
                  for j in range(4)]
            qi = (qb[0].astype(jnp.int32)
                  | (qb[1].astype(jnp.int32) << 8)
                  | (qb[2].astype(jnp.int32) << 16)
                  | (qb[3].astype(jnp.int32) << 24))
            q = jax.lax.bitcast_convert_type(qi, jnp.float32)
            residual[p] = residual[p] - q
            total[p] = total[p] + q
    for p in range(_P):
        out_ref[p * _TP:(p + 1) * _TP] = zs[p] + (total[p] - zs[p])


def kernel(z, codebooks, W1, b1, W2, b2):
    B, N, D = z.shape
    nt = B * N

    cb2, planes, c2 = pl.pallas_call(
        _mlp_body,
        out_shape=[
            jax.ShapeDtypeStruct((_S, _K, _D), jnp.float32),
            jax.ShapeDtypeStruct((_S, 4, _K, _D), jnp.bfloat16),
            jax.ShapeDtypeStruct((_S, 1, _K), jnp.float32),
        ],
    )(codebooks, W1, b1.reshape(_S, 1, _D), W2, b2.reshape(_S, 1, _D))

    out = pl.pallas_call(
        _fused_body,
        grid=(nt // _T,),
        in_specs=[
            pl.BlockSpec((_T, _D), lambda t: (t, 0)),
            pl.BlockSpec((_S, _K, _D), lambda t: (0, 0, 0)),
            pl.BlockSpec((_S, 1, _K), lambda t: (0, 0, 0)),
            pl.BlockSpec((_S, 4, _K, _D), lambda t: (0, 0, 0, 0)),
        ],
        out_specs=pl.BlockSpec((_T, _D), lambda t: (t, 0)),
        out_shape=jax.ShapeDtypeStruct((nt, D), jnp.float32),
    )(z.reshape(nt, D), cb2, c2, planes)
    return out.reshape(B, N, D)


# speedup vs baseline: 1.2776x; 1.0043x over previous
import jax
import jax.numpy as jnp
from jax.experimental import pallas as pl
from jax.experimental.pallas import tpu as pltpu

_S = 4
_K = 1024
_D = 256
_T = 2048


def _mlp_body(cb_ref, w1_ref, b1_ref, w2_ref, b2_ref,
              cb2_ref, bytes_ref, c2_ref):
    for i in range(_S):
        cb = cb_ref[i]
        h = jnp.maximum(jnp.dot(cb, w1_ref[i]) + b1_ref[i], 0.0)
        cb_eff = jnp.dot(h, w2_ref[i]) + b2_ref[i]
        cb2_ref[i] = cb_eff + cb_eff
        c2_ref[i] = jnp.sum(cb_eff * cb_eff, axis=-1)[None, :]
        u = jax.lax.bitcast_convert_type(cb_eff, jnp.int32)
        for j in range(4):
            scale = jnp.bfloat16(256.0 if j % 2 else 1.0)
            plane = ((u >> (8 * j)) & 0xFF).astype(jnp.bfloat16) * scale
            bytes_ref[i, :, j * _D:(j + 1) * _D] = plane


_P = 4
_TP = _T // _P


def _fused_body(z_ref, cb_ref, c2_ref, bytes_ref, out_ref):
    lanes = jax.lax.broadcasted_iota(jnp.int32, (_TP, _K), 1)
    zs = [z_ref[p * _TP:(p + 1) * _TP] for p in range(_P)]
    residual = list(zs)
    total = [jnp.zeros_like(zs[p]) for p in range(_P)]
    for i in range(_S):
        for p in range(_P):
            r2 = jnp.sum(residual[p] * residual[p],
                         axis=-1, keepdims=True)
            e2 = jax.lax.dot_general(residual[p], cb_ref[i],
                                     (((1,), (1,)), ((), ())))
            d = (r2 - e2) + c2_ref[i]
            mn = jnp.min(d, axis=-1, keepdims=True)
            idx = jnp.min(jnp.where(d == mn, lanes, _K), axis=-1)
            onehot = (lanes == idx[:, None]).astype(jnp.bfloat16)
            qcat = jax.lax.dot_general(onehot, bytes_ref[i],
                                       (((1,), (0,)), ((), ())),
                                       preferred_element_type=jnp.float32)
            lo = qcat[:, 0:_D] + qcat[:, _D:2 * _D]
            hi = qcat[:, 2 * _D:3 * _D] + qcat[:, 3 * _D:4 * _D]
            qi = lo.astype(jnp.int32) | (hi.astype(jnp.int32) << 16)
            q = jax.lax.bitcast_convert_type(qi, jnp.float32)
            residual[p] = residual[p] - q
            total[p] = total[p] + q
    for p in range(_P):
        out_ref[p * _TP:(p + 1) * _TP] = zs[p] + (total[p] - zs[p])


def kernel(z, codebooks, W1, b1, W2, b2):
    B, N, D = z.shape
    nt = B * N

    cb2, planes, c2 = pl.pallas_call(
        _mlp_body,
        out_shape=[
            jax.ShapeDtypeStruct((_S, _K, _D), jnp.float32),
            jax.ShapeDtypeStruct((_S, _K, 4 * _D), jnp.bfloat16),
            jax.ShapeDtypeStruct((_S, 1, _K), jnp.float32),
        ],
    )(codebooks, W1, b1.reshape(_S, 1, _D), W2, b2.reshape(_S, 1, _D))

    out = pl.pallas_call(
        _fused_body,
        grid=(nt // _T,),
        in_specs=[
            pl.BlockSpec((_T, _D), lambda t: (t, 0)),
            pl.BlockSpec((_S, _K, _D), lambda t: (0, 0, 0)),
            pl.BlockSpec((_S, 1, _K), lambda t: (0, 0, 0)),
            pl.BlockSpec((_S, _K, 4 * _D), lambda t: (0, 0, 0)),
        ],
        out_specs=pl.BlockSpec((_T, _D), lambda t: (t, 0)),
        out_shape=jax.ShapeDtypeStruct((nt, D), jnp.float32),
    )(z.reshape(nt, D), cb2, c2, planes)
    return out.reshape(B, N, D)


# speedup vs baseline: 1.3988x; 1.0949x over previous
import jax
import jax.numpy as jnp
from jax.experimental import pallas as pl
from jax.experimental.pallas import tpu as pltpu

_S = 4
_K = 1024
_D = 256
_T = 2048


def _mlp_body(cb_ref, w1_ref, b1_ref, w2_ref, b2_ref,
              cb2_ref, bytes_ref, c2_ref):
    for i in range(_S):
        cb = cb_ref[i]
        h = jnp.maximum(jnp.dot(cb, w1_ref[i]) + b1_ref[i], 0.0)
        cb_eff = jnp.dot(h, w2_ref[i]) + b2_ref[i]
        cb2_ref[i] = cb_eff + cb_eff
        c2_ref[i] = jnp.sum(cb_eff * cb_eff, axis=-1)[None, :]
        u = jax.lax.bitcast_convert_type(cb_eff, jnp.int32)
        for j in range(4):
            scale = jnp.bfloat16(256.0 if j % 2 else 1.0)
            plane = ((u >> (8 * j)) & 0xFF).astype(jnp.bfloat16) * scale
            bytes_ref[i, :, j * _D:(j + 1) * _D] = plane


_P = 2
_TP = _T // _P


def _fused_body(z_ref, cb_ref, c2_ref, bytes_ref, out_ref):
    lanes = jax.lax.broadcasted_iota(jnp.int32, (_TP, _K), 1)
    zs = [z_ref[p * _TP:(p + 1) * _TP] for p in range(_P)]
    residual = list(zs)
    total = [jnp.zeros_like(zs[p]) for p in range(_P)]
    for i in range(_S):
        for p in range(_P):
            r2 = jnp.sum(residual[p] * residual[p],
                         axis=-1, keepdims=True)
            e2 = jax.lax.dot_general(residual[p], cb_ref[i],
                                     (((1,), (1,)), ((), ())))
            d = (r2 - e2) + c2_ref[i]
            mn = jnp.min(d, axis=-1, keepdims=True)
            idx = jnp.min(jnp.where(d == mn, lanes, _K), axis=-1)
            onehot = (lanes == idx[:, None]).astype(jnp.bfloat16)
            qcat = jax.lax.dot_general(onehot, bytes_ref[i],
                                       (((1,), (0,)), ((), ())),
                                       preferred_element_type=jnp.float32)
            lo = qcat[:, 0:_D] + qcat[:, _D:2 * _D]
            hi = qcat[:, 2 * _D:3 * _D] + qcat[:, 3 * _D:4 * _D]
            qi = lo.astype(jnp.int32) | (hi.astype(jnp.int32) << 16)
            q = jax.lax.bitcast_convert_type(qi, jnp.float32)
            residual[p] = residual[p] - q
            total[p] = total[p] + q
    for p in range(_P):
        out_ref[p * _TP:(p + 1) * _TP] = zs[p] + (total[p] - zs[p])


def kernel(z, codebooks, W1, b1, W2, b2):
    B, N, D = z.shape
    nt = B * N

    cb2, planes, c2 = pl.pallas_call(
        _mlp_body,
        out_shape=[
            jax.ShapeDtypeStruct((_S, _K, _D), jnp.float32),
            jax.ShapeDtypeStruct((_S, _K, 4 * _D), jnp.bfloat16),
            jax.ShapeDtypeStruct((_S, 1, _K), jnp.float32),
        ],
    )(codebooks, W1, b1.reshape(_S, 1, _D), W2, b2.reshape(_S, 1, _D))

    out = pl.pallas_call(
        _fused_body,
        grid=(nt // _T,),
        in_specs=[
            pl.BlockSpec((_T, _D), lambda t: (t, 0)),
            pl.BlockSpec((_S, _K, _D), lambda t: (0, 0, 0)),
            pl.BlockSpec((_S, 1, _K), lambda t: (0, 0, 0)),
            pl.BlockSpec((_S, _K, 4 * _D), lambda t: (0, 0, 0)),
        ],
        out_specs=pl.BlockSpec((_T, _D), lambda t: (t, 0)),
        out_shape=jax.ShapeDtypeStruct((nt, D), jnp.float32),
    )(z.reshape(nt, D), cb2, c2, planes)
    return out.reshape(B, N, D)
